# baseline (speedup 1.0000x reference)
import functools

import jax
import jax.numpy as jnp
from jax import lax
from jax.experimental import pallas as pl
from jax.experimental.pallas import tpu as pltpu
from jax.experimental.pallas import tpu_sc as plsc

NC = 2
NS = 16
LANES = 16
CH = 80


def _sc_aggregate(x0, x1, src3, dst3, w3):
    N, Dh = x0.shape
    ns, rows_per_tile, ch = src3.shape
    n_out_blocks = N // ch
    blocks_per_tile = (n_out_blocks + NS - 1) // NS
    mesh = plsc.VectorSubcoreMesh(core_axis_name="c", subcore_axis_name="s")

    @functools.partial(
        pl.kernel,
        out_type=jax.ShapeDtypeStruct((NC * N, Dh), jnp.float32),
        mesh=mesh,
        compiler_params=pltpu.CompilerParams(use_tc_tiling_on_sc=False),
        scratch_types=[
            pltpu.VMEM((rows_per_tile, ch), jnp.int32),
            pltpu.VMEM((rows_per_tile, ch), jnp.int32),
            pltpu.VMEM((rows_per_tile, ch), jnp.float32),
            pltpu.VMEM((ch, Dh), jnp.float32),
            pltpu.VMEM((ch, Dh), jnp.float32),
            pltpu.VMEM_SHARED((N, Dh), jnp.float32),
            pltpu.SemaphoreType.DMA,
            pltpu.SemaphoreType.DMA,
            pltpu.SemaphoreType.DMA,
            pltpu.SemaphoreType.DMA,
        ],
    )
    def k(x0_hbm, x1_hbm, src_hbm, dst_hbm, w_hbm, out_hbm, src_v, dst_v, w_v,
          rows_a, rows_b, acc_sh, sem_a, sem_b, ssem_a, ssem_b):
        cid = lax.axis_index("c")
        sid = lax.axis_index("s")

        pltpu.sync_copy(src_hbm.at[sid], src_v)
        pltpu.sync_copy(dst_hbm.at[sid], dst_v)
        pltpu.sync_copy(w_hbm.at[sid], w_v)

        @pl.loop(0, ch)
        def _(e):
            for kk in range(Dh // LANES):
                rows_a[e, pl.ds(kk * LANES, LANES)] = jnp.zeros(
                    (LANES,), jnp.float32)

        @pl.loop(0, blocks_per_tile)
        def _(i):
            blk = sid + NS * i

            @pl.when(blk < n_out_blocks)
            def _():
                pltpu.sync_copy(rows_a, acc_sh.at[pl.ds(blk * ch, ch)])

        def gather_start(j, buf, sem):
            @pl.when(cid == 0)
            def _():
                pltpu.async_copy(x0_hbm.at[src_v.at[j]], buf, sem)

            @pl.when(cid == 1)
            def _():
                pltpu.async_copy(x1_hbm.at[src_v.at[j]], buf, sem)

        def gather_wait(j, buf, sem):
            pltpu.make_async_copy(x0_hbm.at[src_v.at[j]], buf, sem).wait()

        def scale(j, buf):
            @pl.loop(0, ch, step=LANES)
            def _(e0):
                wvec = w_v[j, pl.ds(e0, LANES)]
                for i in range(LANES):
                    wb = lax.broadcast(wvec[i], (LANES,))
                    for kk in range(Dh // LANES):
                        sl = (e0 + i, pl.ds(kk * LANES, LANES))
                        buf[sl] = buf[sl] * wb

        def scatter_start(j, buf, sem):
            pltpu.async_copy(buf, acc_sh.at[dst_v.at[j]], sem, add=True)

        def scatter_wait(j, buf, sem):
            pltpu.make_async_copy(buf, acc_sh.at[dst_v.at[j]], sem).wait()

        gather_start(0, rows_a, sem_a)
        plsc.subcore_barrier()

        @pl.loop(0, rows_per_tile, step=2)
        def _(j):
            gather_wait(j, rows_a, sem_a)

            @pl.when(j > 0)
            def _():
                scatter_wait(j - 1, rows_b, ssem_b)

            gather_start(j + 1, rows_b, sem_b)
            scale(j, rows_a)
            scatter_start(j, rows_a, ssem_a)

            gather_wait(j + 1, rows_b, sem_b)
            scale(j + 1, rows_b)
            scatter_wait(j, rows_a, ssem_a)

            @pl.when(j + 2 < rows_per_tile)
            def _():
                gather_start(j + 2, rows_a, sem_a)

            scatter_start(j + 1, rows_b, ssem_b)

        scatter_wait(rows_per_tile - 1, rows_b, ssem_b)
        plsc.subcore_barrier()

        @pl.loop(0, blocks_per_tile)
        def _(i):
            blk = sid + NS * i

            @pl.when(blk < n_out_blocks)
            def _():
                pltpu.sync_copy(
                    acc_sh.at[pl.ds(blk * ch, ch)],
                    out_hbm.at[pl.ds(cid * N + blk * ch, ch)])

    return k(x0, x1, src3, dst3, w3)


def _tc_combine_matmul(agg, W, b):
    two_n, dh = agg.shape
    n = two_n // 2
    d_out = W.shape[1]
    blk = 1000
    grid = n // blk

    def body(a0_ref, a1_ref, w0_ref, w1_ref, b_ref, o_ref):
        o_ref[...] = (
            jnp.dot(a0_ref[...], w0_ref[...],
                    preferred_element_type=jnp.float32)
            + jnp.dot(a1_ref[...], w1_ref[...],
                      preferred_element_type=jnp.float32)
            + b_ref[...]
        )

    return pl.pallas_call(
        body,
        grid=(grid,),
        in_specs=[
            pl.BlockSpec((blk, dh), lambda i: (i, 0)),
            pl.BlockSpec((blk, dh), lambda i: (i, 0)),
            pl.BlockSpec((dh, d_out), lambda i: (0, 0)),
            pl.BlockSpec((dh, d_out), lambda i: (0, 0)),
            pl.BlockSpec((1, d_out), lambda i: (0, 0)),
        ],
        out_specs=pl.BlockSpec((blk, d_out), lambda i: (i, 0)),
        out_shape=jax.ShapeDtypeStruct((n, d_out), jnp.float32),
    )(agg[:n], agg[n:], W[:dh], W[dh:], b.reshape(1, d_out))


@jax.jit
def kernel(x, edge_index, edge_weight, W, b):
    e = edge_weight.shape[0]
    d = x.shape[1]
    dh = d // 2
    rows_per_tile = e // (NS * CH)
    shape3 = (NS, rows_per_tile, CH)
    src3 = edge_index[0].reshape(shape3)
    dst3 = edge_index[1].reshape(shape3)
    w3 = edge_weight.reshape(shape3)
    x0 = x[:, :dh]
    x1 = x[:, dh:]
    agg = _sc_aggregate(x0, x1, src3, dst3, w3)
    return _tc_combine_matmul(agg, W, b)


# speedup vs baseline: 1.0411x; 1.0411x over previous
import functools

import jax
import jax.numpy as jnp
from jax import lax
from jax.experimental import pallas as pl
from jax.experimental.pallas import tpu as pltpu
from jax.experimental.pallas import tpu_sc as plsc

NC = 2
NS = 16
LANES = 16
CH = 128
ZB = 80


def _sc_aggregate(x0, x1, src3, dst3, w3):
    N, Dh = x0.shape
    ns, rows_per_tile, ch = src3.shape
    n_out_blocks = N // ZB
    blocks_per_tile = (n_out_blocks + NS - 1) // NS
    mesh = plsc.VectorSubcoreMesh(core_axis_name="c", subcore_axis_name="s")

    @functools.partial(
        pl.kernel,
        out_type=jax.ShapeDtypeStruct((NC * N, Dh), jnp.float32),
        mesh=mesh,
        compiler_params=pltpu.CompilerParams(use_tc_tiling_on_sc=False),
        scratch_types=[
            pltpu.VMEM((rows_per_tile, ch), jnp.int32),
            pltpu.VMEM((rows_per_tile, ch), jnp.int32),
            pltpu.VMEM((rows_per_tile, ch), jnp.float32),
            pltpu.VMEM((ch, Dh), jnp.float32),
            pltpu.VMEM((ch, Dh), jnp.float32),
            pltpu.VMEM_SHARED((N, Dh), jnp.float32),
            pltpu.SemaphoreType.DMA,
            pltpu.SemaphoreType.DMA,
        ],
    )
    def k(x0_hbm, x1_hbm, src_hbm, dst_hbm, w_hbm, out_hbm, src_v, dst_v, w_v,
          rows_a, rows_b, acc_sh, sem_a, sem_b):
        cid = lax.axis_index("c")
        sid = lax.axis_index("s")

        pltpu.sync_copy(src_hbm.at[sid], src_v)
        pltpu.sync_copy(dst_hbm.at[sid], dst_v)
        pltpu.sync_copy(w_hbm.at[sid], w_v)

        @pl.loop(0, ch)
        def _(e):
            for kk in range(Dh // LANES):
                rows_a[e, pl.ds(kk * LANES, LANES)] = jnp.zeros(
                    (LANES,), jnp.float32)

        @pl.loop(0, blocks_per_tile)
        def _(i):
            blk = sid + NS * i

            @pl.when(blk < n_out_blocks)
            def _():
                pltpu.sync_copy(rows_a.at[pl.ds(0, ZB)],
                                acc_sh.at[pl.ds(blk * ZB, ZB)])

        def gather_start(j, buf, sem):
            @pl.when(cid == 0)
            def _():
                pltpu.async_copy(x0_hbm.at[src_v.at[j]], buf, sem)

            @pl.when(cid == 1)
            def _():
                pltpu.async_copy(x1_hbm.at[src_v.at[j]], buf, sem)

        def gather_wait(j, buf, sem):
            pltpu.make_async_copy(x0_hbm.at[src_v.at[j]], buf, sem).wait()

        def scale(j, buf):
            @pl.loop(0, ch, step=LANES)
            def _(e0):
                wvec = w_v[j, pl.ds(e0, LANES)]
                for i in range(LANES):
                    wb = lax.broadcast(wvec[i], (LANES,))
                    for kk in range(Dh // LANES):
                        sl = (e0 + i, pl.ds(kk * LANES, LANES))
                        buf[sl] = buf[sl] * wb

        gather_start(0, rows_a, sem_a)
        plsc.subcore_barrier()

        @pl.loop(0, rows_per_tile, step=2)
        def _(j):
            gather_wait(j, rows_a, sem_a)
            gather_start(j + 1, rows_b, sem_b)
            scale(j, rows_a)
            pltpu.sync_copy(rows_a, acc_sh.at[dst_v.at[j]], add=True)

            gather_wait(j + 1, rows_b, sem_b)

            @pl.when(j + 2 < rows_per_tile)
            def _():
                gather_start(j + 2, rows_a, sem_a)

            scale(j + 1, rows_b)
            pltpu.sync_copy(rows_b, acc_sh.at[dst_v.at[j + 1]], add=True)

        plsc.subcore_barrier()

        @pl.loop(0, blocks_per_tile)
        def _(i):
            blk = sid + NS * i

            @pl.when(blk < n_out_blocks)
            def _():
                pltpu.sync_copy(
                    acc_sh.at[pl.ds(blk * ZB, ZB)],
                    out_hbm.at[pl.ds(cid * N + blk * ZB, ZB)])

    return k(x0, x1, src3, dst3, w3)


def _tc_combine_matmul(agg, W, b):
    two_n, dh = agg.shape
    n = two_n // 2
    d_out = W.shape[1]
    blk = 1000
    grid = n // blk

    def body(a0_ref, a1_ref, w0_ref, w1_ref, b_ref, o_ref):
        o_ref[...] = (
            jnp.dot(a0_ref[...], w0_ref[...],
                    preferred_element_type=jnp.float32)
            + jnp.dot(a1_ref[...], w1_ref[...],
                      preferred_element_type=jnp.float32)
            + b_ref[...]
        )

    return pl.pallas_call(
        body,
        grid=(grid,),
        in_specs=[
            pl.BlockSpec((blk, dh), lambda i: (i, 0)),
            pl.BlockSpec((blk, dh), lambda i: (i, 0)),
            pl.BlockSpec((dh, d_out), lambda i: (0, 0)),
            pl.BlockSpec((dh, d_out), lambda i: (0, 0)),
            pl.BlockSpec((1, d_out), lambda i: (0, 0)),
        ],
        out_specs=pl.BlockSpec((blk, d_out), lambda i: (i, 0)),
        out_shape=jax.ShapeDtypeStruct((n, d_out), jnp.float32),
    )(agg[:n], agg[n:], W[:dh], W[dh:], b.reshape(1, d_out))


@jax.jit
def kernel(x, edge_index, edge_weight, W, b):
    e = edge_weight.shape[0]
    dh = x.shape[1] // 2
    rows_per_tile = -(-e // (NS * CH))
    if rows_per_tile % 2:
        rows_per_tile += 1
    e_pad = NS * CH * rows_per_tile
    pad = e_pad - e
    src = jnp.concatenate([edge_index[0], jnp.zeros((pad,), jnp.int32)])
    dst = jnp.concatenate([edge_index[1], jnp.zeros((pad,), jnp.int32)])
    w = jnp.concatenate([edge_weight, jnp.zeros((pad,), jnp.float32)])
    shape3 = (NS, rows_per_tile, CH)
    agg = _sc_aggregate(x[:, :dh], x[:, dh:], src.reshape(shape3),
                        dst.reshape(shape3), w.reshape(shape3))
    return _tc_combine_matmul(agg, W, b)


# speedup vs baseline: 1.7548x; 1.6855x over previous
import functools

import jax
import jax.numpy as jnp
from jax import lax
from jax.experimental import pallas as pl
from jax.experimental.pallas import tpu as pltpu
from jax.experimental.pallas import tpu_sc as plsc

NC = 2
NS = 16
LANES = 16
CH = 80
ZB = 80


def _sc_aggregate(x0, x1, src3, dst3, w3):
    N, Dh = x0.shape
    ns, rows_per_tile, ch = src3.shape
    n_out_blocks = N // ZB
    blocks_per_tile = (n_out_blocks + NS - 1) // NS
    mesh = plsc.VectorSubcoreMesh(core_axis_name="c", subcore_axis_name="s")

    @functools.partial(
        pl.kernel,
        out_type=jax.ShapeDtypeStruct((NC * N, Dh), jnp.float32),
        mesh=mesh,
        compiler_params=pltpu.CompilerParams(use_tc_tiling_on_sc=False),
        scratch_types=[
            pltpu.VMEM((rows_per_tile, ch), jnp.int32),
            pltpu.VMEM((rows_per_tile, ch), jnp.int32),
            pltpu.VMEM((rows_per_tile, ch), jnp.float32),
            pltpu.VMEM((ch, Dh), jnp.float32),
            pltpu.VMEM((ch, Dh), jnp.float32),
            pltpu.VMEM_SHARED((N, Dh), jnp.float32),
            pltpu.SemaphoreType.DMA,
            pltpu.SemaphoreType.DMA,
        ],
    )
    def k(x0_hbm, x1_hbm, src_hbm, dst_hbm, w_hbm, out_hbm, src_v, dst_v, w_v,
          rows_a, rows_b, acc_sh, sem_a, sem_b):
        cid = lax.axis_index("c")
        sid = lax.axis_index("s")

        pltpu.sync_copy(src_hbm.at[sid], src_v)
        pltpu.sync_copy(dst_hbm.at[sid], dst_v)
        pltpu.sync_copy(w_hbm.at[sid], w_v)

        @pl.loop(0, ch)
        def _(e):
            for kk in range(Dh // LANES):
                rows_a[e, pl.ds(kk * LANES, LANES)] = jnp.zeros(
                    (LANES,), jnp.float32)

        @pl.loop(0, blocks_per_tile)
        def _(i):
            blk = sid + NS * i

            @pl.when(blk < n_out_blocks)
            def _():
                pltpu.sync_copy(rows_a.at[pl.ds(0, ZB)],
                                acc_sh.at[pl.ds(blk * ZB, ZB)])

        def gather_start(j, buf, sem):
            @pl.when(cid == 0)
            def _():
                pltpu.async_copy(x0_hbm.at[src_v.at[j]], buf, sem)

            @pl.when(cid == 1)
            def _():
                pltpu.async_copy(x1_hbm.at[src_v.at[j]], buf, sem)

        def gather_wait(j, buf, sem):
            pltpu.make_async_copy(x0_hbm.at[src_v.at[j]], buf, sem).wait()

        def scale(j, buf):
            @plsc.parallel_loop(0, ch, step=LANES)
            def _(e0):
                wvec = w_v[j, pl.ds(e0, LANES)]
                for i in range(LANES):
                    wb = lax.broadcast(wvec[i], (LANES,))
                    for kk in range(Dh // LANES):
                        sl = (e0 + i, pl.ds(kk * LANES, LANES))
                        buf[sl] = buf[sl] * wb

        gather_start(0, rows_a, sem_a)
        plsc.subcore_barrier()

        @pl.loop(0, rows_per_tile, step=2)
        def _(j):
            gather_wait(j, rows_a, sem_a)
            gather_start(j + 1, rows_b, sem_b)
            scale(j, rows_a)
            pltpu.sync_copy(rows_a, acc_sh.at[dst_v.at[j]], add=True)

            gather_wait(j + 1, rows_b, sem_b)

            @pl.when(j + 2 < rows_per_tile)
            def _():
                gather_start(j + 2, rows_a, sem_a)

            scale(j + 1, rows_b)
            pltpu.sync_copy(rows_b, acc_sh.at[dst_v.at[j + 1]], add=True)

        plsc.subcore_barrier()

        @pl.loop(0, blocks_per_tile)
        def _(i):
            blk = sid + NS * i

            @pl.when(blk < n_out_blocks)
            def _():
                pltpu.sync_copy(
                    acc_sh.at[pl.ds(blk * ZB, ZB)],
                    out_hbm.at[pl.ds(cid * N + blk * ZB, ZB)])

    return k(x0, x1, src3, dst3, w3)


def _tc_combine_matmul(agg, W, b):
    two_n, dh = agg.shape
    n = two_n // 2
    d_out = W.shape[1]
    blk = 1000
    grid = n // blk

    def body(a0_ref, a1_ref, w0_ref, w1_ref, b_ref, o_ref):
        o_ref[...] = (
            jnp.dot(a0_ref[...], w0_ref[...],
                    preferred_element_type=jnp.float32)
            + jnp.dot(a1_ref[...], w1_ref[...],
                      preferred_element_type=jnp.float32)
            + b_ref[...]
        )

    return pl.pallas_call(
        body,
        grid=(grid,),
        in_specs=[
            pl.BlockSpec((blk, dh), lambda i: (i, 0)),
            pl.BlockSpec((blk, dh), lambda i: (i, 0)),
            pl.BlockSpec((dh, d_out), lambda i: (0, 0)),
            pl.BlockSpec((dh, d_out), lambda i: (0, 0)),
            pl.BlockSpec((1, d_out), lambda i: (0, 0)),
        ],
        out_specs=pl.BlockSpec((blk, d_out), lambda i: (i, 0)),
        out_shape=jax.ShapeDtypeStruct((n, d_out), jnp.float32),
    )(agg[:n], agg[n:], W[:dh], W[dh:], b.reshape(1, d_out))


@jax.jit
def kernel(x, edge_index, edge_weight, W, b):
    e = edge_weight.shape[0]
    dh = x.shape[1] // 2
    rows_per_tile = -(-e // (NS * CH))
    if rows_per_tile % 2:
        rows_per_tile += 1
    e_pad = NS * CH * rows_per_tile
    pad = e_pad - e
    src = jnp.concatenate([edge_index[0], jnp.zeros((pad,), jnp.int32)])
    dst = jnp.concatenate([edge_index[1], jnp.zeros((pad,), jnp.int32)])
    w = jnp.concatenate([edge_weight, jnp.zeros((pad,), jnp.float32)])
    shape3 = (NS, rows_per_tile, CH)
    agg = _sc_aggregate(x[:, :dh], x[:, dh:], src.reshape(shape3),
                        dst.reshape(shape3), w.reshape(shape3))
    return _tc_combine_matmul(agg, W, b)


# speedup vs baseline: 2.1082x; 1.2014x over previous
import functools

import jax
import jax.numpy as jnp
from jax import lax
from jax.experimental import pallas as pl
from jax.experimental.pallas import tpu as pltpu
from jax.experimental.pallas import tpu_sc as plsc

NC = 2
NS = 16
LANES = 16
CH = 80
ZB = 80


def _sc_aggregate(x0, x1, src3, dst3, w3):
    N, Dh = x0.shape
    ns, rows_per_tile, ch = src3.shape
    n_out_blocks = N // ZB
    blocks_per_tile = (n_out_blocks + NS - 1) // NS
    mesh = plsc.VectorSubcoreMesh(core_axis_name="c", subcore_axis_name="s")

    @functools.partial(
        pl.kernel,
        out_type=jax.ShapeDtypeStruct((NC * N, Dh), jnp.float32),
        mesh=mesh,
        compiler_params=pltpu.CompilerParams(use_tc_tiling_on_sc=False),
        scratch_types=[
            pltpu.VMEM((rows_per_tile, ch), jnp.int32),
            pltpu.VMEM((rows_per_tile, ch), jnp.int32),
            pltpu.VMEM((rows_per_tile, ch), jnp.float32),
            pltpu.VMEM((ch, Dh), jnp.float32),
            pltpu.VMEM((ch, Dh), jnp.float32),
            pltpu.VMEM((ch, Dh), jnp.float32),
            pltpu.VMEM((ch, Dh), jnp.float32),
            pltpu.VMEM_SHARED((N, Dh), jnp.float32),
            pltpu.SemaphoreType.DMA,
            pltpu.SemaphoreType.DMA,
            pltpu.SemaphoreType.DMA,
            pltpu.SemaphoreType.DMA,
        ],
    )
    def k(x0_hbm, x1_hbm, src_hbm, dst_hbm, w_hbm, out_hbm, src_v, dst_v, w_v,
          rows_a, rows_b, rows_c, rows_d, acc_sh, sem_a, sem_b, sem_c, sem_d):
        cid = lax.axis_index("c")
        sid = lax.axis_index("s")

        pltpu.sync_copy(src_hbm.at[sid], src_v)
        pltpu.sync_copy(dst_hbm.at[sid], dst_v)
        pltpu.sync_copy(w_hbm.at[sid], w_v)

        @pl.loop(0, ch)
        def _(e):
            for kk in range(Dh // LANES):
                rows_a[e, pl.ds(kk * LANES, LANES)] = jnp.zeros(
                    (LANES,), jnp.float32)

        @pl.loop(0, blocks_per_tile)
        def _(i):
            blk = sid + NS * i

            @pl.when(blk < n_out_blocks)
            def _():
                pltpu.sync_copy(rows_a.at[pl.ds(0, ZB)],
                                acc_sh.at[pl.ds(blk * ZB, ZB)])

        def gather_start(j, buf, sem):
            @pl.when(cid == 0)
            def _():
                pltpu.async_copy(x0_hbm.at[src_v.at[j]], buf, sem)

            @pl.when(cid == 1)
            def _():
                pltpu.async_copy(x1_hbm.at[src_v.at[j]], buf, sem)

        def gather_wait(j, buf, sem):
            pltpu.make_async_copy(x0_hbm.at[src_v.at[j]], buf, sem).wait()

        def scale(j, buf):
            @plsc.parallel_loop(0, ch, step=LANES)
            def _(e0):
                wvec = w_v[j, pl.ds(e0, LANES)]
                for i in range(LANES):
                    wb = lax.broadcast(wvec[i], (LANES,))
                    for kk in range(Dh // LANES):
                        sl = (e0 + i, pl.ds(kk * LANES, LANES))
                        buf[sl] = buf[sl] * wb

        bufs = (rows_a, rows_b, rows_c, rows_d)
        sems = (sem_a, sem_b, sem_c, sem_d)
        gather_start(0, bufs[0], sems[0])
        gather_start(1, bufs[1], sems[1])
        gather_start(2, bufs[2], sems[2])
        plsc.subcore_barrier()

        @pl.loop(0, rows_per_tile, step=4)
        def _(j):
            for k in range(4):
                jk = j + k
                buf, sem = bufs[k], sems[k]
                gather_wait(jk, buf, sem)
                nxt = jk + 3
                nbuf, nsem = bufs[(k + 3) % 4], sems[(k + 3) % 4]
                if k == 0:
                    gather_start(nxt, nbuf, nsem)
                else:
                    @pl.when(nxt < rows_per_tile)
                    def _():
                        gather_start(nxt, nbuf, nsem)

                scale(jk, buf)
                pltpu.sync_copy(buf, acc_sh.at[dst_v.at[jk]], add=True)

        plsc.subcore_barrier()

        @pl.loop(0, blocks_per_tile)
        def _(i):
            blk = sid + NS * i

            @pl.when(blk < n_out_blocks)
            def _():
                pltpu.sync_copy(
                    acc_sh.at[pl.ds(blk * ZB, ZB)],
                    out_hbm.at[pl.ds(cid * N + blk * ZB, ZB)])

    return k(x0, x1, src3, dst3, w3)


def _tc_combine_matmul(agg, W, b):
    two_n, dh = agg.shape
    n = two_n // 2
    d_out = W.shape[1]
    blk = 1000
    grid = n // blk

    def body(a0_ref, a1_ref, w0_ref, w1_ref, b_ref, o_ref):
        o_ref[...] = (
            jnp.dot(a0_ref[...], w0_ref[...],
                    preferred_element_type=jnp.float32)
            + jnp.dot(a1_ref[...], w1_ref[...],
                      preferred_element_type=jnp.float32)
            + b_ref[...]
        )

    return pl.pallas_call(
        body,
        grid=(grid,),
        in_specs=[
            pl.BlockSpec((blk, dh), lambda i: (i, 0)),
            pl.BlockSpec((blk, dh), lambda i: (i, 0)),
            pl.BlockSpec((dh, d_out), lambda i: (0, 0)),
            pl.BlockSpec((dh, d_out), lambda i: (0, 0)),
            pl.BlockSpec((1, d_out), lambda i: (0, 0)),
        ],
        out_specs=pl.BlockSpec((blk, d_out), lambda i: (i, 0)),
        out_shape=jax.ShapeDtypeStruct((n, d_out), jnp.float32),
    )(agg[:n], agg[n:], W[:dh], W[dh:], b.reshape(1, d_out))


@jax.jit
def kernel(x, edge_index, edge_weight, W, b):
    e = edge_weight.shape[0]
    dh = x.shape[1] // 2
    rows_per_tile = -(-e // (NS * CH))
    rows_per_tile = -(-rows_per_tile // 4) * 4
    e_pad = NS * CH * rows_per_tile
    pad = e_pad - e
    src = jnp.concatenate([edge_index[0], jnp.zeros((pad,), jnp.int32)])
    dst = jnp.concatenate([edge_index[1], jnp.zeros((pad,), jnp.int32)])
    w = jnp.concatenate([edge_weight, jnp.zeros((pad,), jnp.float32)])
    shape3 = (NS, rows_per_tile, CH)
    agg = _sc_aggregate(x[:, :dh], x[:, dh:], src.reshape(shape3),
                        dst.reshape(shape3), w.reshape(shape3))
    return _tc_combine_matmul(agg, W, b)


# speedup vs baseline: 2.5643x; 1.2163x over previous
import functools

import jax
import jax.numpy as jnp
from jax import lax
from jax.experimental import pallas as pl
from jax.experimental.pallas import tpu as pltpu
from jax.experimental.pallas import tpu_sc as plsc

NC = 2
NS = 16
LANES = 16
CH = 80
ZB = 80
NBUF = 5


def _sc_aggregate(x0, x1, src3, dst3, w3):
    N, Dh = x0.shape
    ns, rows_per_tile, ch = src3.shape
    n_out_blocks = N // ZB
    blocks_per_tile = (n_out_blocks + NS - 1) // NS
    mesh = plsc.VectorSubcoreMesh(core_axis_name="c", subcore_axis_name="s")

    @functools.partial(
        pl.kernel,
        out_type=jax.ShapeDtypeStruct((NC * N, Dh), jnp.float32),
        mesh=mesh,
        compiler_params=pltpu.CompilerParams(use_tc_tiling_on_sc=False),
        scratch_types=[
            pltpu.VMEM((rows_per_tile, ch), jnp.int32),
            pltpu.VMEM((rows_per_tile, ch), jnp.int32),
            pltpu.VMEM((rows_per_tile, ch), jnp.float32),
            *[pltpu.VMEM((ch, Dh), jnp.float32)
              for _ in range(NBUF)],
            pltpu.VMEM_SHARED((N, Dh), jnp.float32),
            *[pltpu.SemaphoreType.DMA for _ in range(NBUF)],
        ],
    )
    def k(x0_hbm, x1_hbm, src_hbm, dst_hbm, w_hbm, out_hbm, src_v, dst_v, w_v,
          *ring):
        bufs = ring[:NBUF]
        acc_sh = ring[NBUF]
        sems = ring[NBUF + 1:NBUF + 1 + NBUF]
        cid = lax.axis_index("c")
        sid = lax.axis_index("s")

        pltpu.sync_copy(src_hbm.at[sid], src_v)
        pltpu.sync_copy(dst_hbm.at[sid], dst_v)
        pltpu.sync_copy(w_hbm.at[sid], w_v)

        rows_a = bufs[0]

        @pl.loop(0, ch)
        def _(e):
            for kk in range(Dh // LANES):
                rows_a[e, pl.ds(kk * LANES, LANES)] = jnp.zeros(
                    (LANES,), jnp.float32)

        @pl.loop(0, blocks_per_tile)
        def _(i):
            blk = sid + NS * i

            @pl.when(blk < n_out_blocks)
            def _():
                pltpu.sync_copy(rows_a.at[pl.ds(0, ZB)],
                                acc_sh.at[pl.ds(blk * ZB, ZB)])

        def gather_start(j, buf, sem):
            @pl.when(cid == 0)
            def _():
                pltpu.async_copy(x0_hbm.at[src_v.at[j]], buf, sem)

            @pl.when(cid == 1)
            def _():
                pltpu.async_copy(x1_hbm.at[src_v.at[j]], buf, sem)

        def gather_wait(j, buf, sem):
            pltpu.make_async_copy(x0_hbm.at[src_v.at[j]], buf, sem).wait()

        def scale(j, buf):
            @plsc.parallel_loop(0, ch, step=LANES)
            def _(e0):
                wvec = w_v[j, pl.ds(e0, LANES)]
                for i in range(LANES):
                    wb = lax.broadcast(wvec[i], (LANES,))
                    for kk in range(Dh // LANES):
                        sl = (e0 + i, pl.ds(kk * LANES, LANES))
                        buf[sl] = buf[sl] * wb

        for k in range(NBUF - 1):
            gather_start(k, bufs[k], sems[k])
        plsc.subcore_barrier()

        @pl.loop(0, rows_per_tile, step=NBUF)
        def _(j):
            for k in range(NBUF):
                jk = j + k
                buf, sem = bufs[k], sems[k]
                gather_wait(jk, buf, sem)
                nxt = jk + NBUF - 1
                nbuf = bufs[(k + NBUF - 1) % NBUF]
                nsem = sems[(k + NBUF - 1) % NBUF]
                if k == 0:
                    gather_start(nxt, nbuf, nsem)
                else:
                    @pl.when(nxt < rows_per_tile)
                    def _():
                        gather_start(nxt, nbuf, nsem)

                scale(jk, buf)
                pltpu.sync_copy(buf, acc_sh.at[dst_v.at[jk]], add=True)

        plsc.subcore_barrier()

        @pl.loop(0, blocks_per_tile)
        def _(i):
            blk = sid + NS * i

            @pl.when(blk < n_out_blocks)
            def _():
                pltpu.sync_copy(
                    acc_sh.at[pl.ds(blk * ZB, ZB)],
                    out_hbm.at[pl.ds(cid * N + blk * ZB, ZB)])

    return k(x0, x1, src3, dst3, w3)


def _tc_combine_matmul(agg, W, b):
    two_n, dh = agg.shape
    n = two_n // 2
    d_out = W.shape[1]
    blk = 1000
    grid = n // blk

    def body(a0_ref, a1_ref, w0_ref, w1_ref, b_ref, o_ref):
        o_ref[...] = (
            jnp.dot(a0_ref[...], w0_ref[...],
                    preferred_element_type=jnp.float32)
            + jnp.dot(a1_ref[...], w1_ref[...],
                      preferred_element_type=jnp.float32)
            + b_ref[...]
        )

    return pl.pallas_call(
        body,
        grid=(grid,),
        in_specs=[
            pl.BlockSpec((blk, dh), lambda i: (i, 0)),
            pl.BlockSpec((blk, dh), lambda i: (i, 0)),
            pl.BlockSpec((dh, d_out), lambda i: (0, 0)),
            pl.BlockSpec((dh, d_out), lambda i: (0, 0)),
            pl.BlockSpec((1, d_out), lambda i: (0, 0)),
        ],
        out_specs=pl.BlockSpec((blk, d_out), lambda i: (i, 0)),
        out_shape=jax.ShapeDtypeStruct((n, d_out), jnp.float32),
    )(agg[:n], agg[n:], W[:dh], W[dh:], b.reshape(1, d_out))


@jax.jit
def kernel(x, edge_index, edge_weight, W, b):
    e = edge_weight.shape[0]
    dh = x.shape[1] // 2
    rows_per_tile = -(-e // (NS * CH))
    rows_per_tile = -(-rows_per_tile // NBUF) * NBUF
    e_pad = NS * CH * rows_per_tile
    pad = e_pad - e
    src = jnp.concatenate([edge_index[0], jnp.zeros((pad,), jnp.int32)])
    dst = jnp.concatenate([edge_index[1], jnp.zeros((pad,), jnp.int32)])
    w = jnp.concatenate([edge_weight, jnp.zeros((pad,), jnp.float32)])
    shape3 = (NS, rows_per_tile, CH)
    agg = _sc_aggregate(x[:, :dh], x[:, dh:], src.reshape(shape3),
                        dst.reshape(shape3), w.reshape(shape3))
    return _tc_combine_matmul(agg, W, b)
